# Initial kernel scaffold; baseline (speedup 1.0000x reference)
import jax
import jax.numpy as jnp
from jax.experimental import pallas as pl


def kernel(x, gate_W, gate_b, fc1_W, fc1_b, ln1_g, ln1_b, res_W, res_b, fc2_W, fc2_b, ln2_g, ln2_b, fc3_W, fc3_b, ln3_g, ln3_b, fc4_W, fc4_b):
    raise NotImplementedError("write your pallas kernel here")


# speedup vs baseline: 2.0406x; 2.0406x over previous
import functools

import jax
import jax.numpy as jnp
from jax.experimental import pallas as pl
from jax.experimental.pallas import tpu as pltpu

_E = 8
_K = 2
_D = 1024
_H = 1024
_F = 512
_O = 1024
_N = 2048
_T = 128
_P = _N * _K + _E * _T


def _ln(t, g, b):
    m = jnp.mean(t, axis=-1, keepdims=True)
    v = jnp.mean((t - m) ** 2, axis=-1, keepdims=True)
    return (t - m) * jax.lax.rsqrt(v + 1e-5) * g + b


def _gelu(t):
    return 0.5 * t * (1.0 + jax.lax.erf(t * 0.7071067811865476))


def _expert_body(gid_ref, x_ref, w_ref,
                 fc1_ref, fc1b_ref, ln1g_ref, ln1b_ref,
                 res_ref, resb_ref,
                 fc2_ref, fc2b_ref, ln2g_ref, ln2b_ref,
                 fc3_ref, fc3b_ref, ln3g_ref, ln3b_ref,
                 fc4_ref, fc4b_ref,
                 out_ref):
    x = x_ref[...]
    h = jnp.dot(x, fc1_ref[0], preferred_element_type=jnp.float32) + fc1b_ref[0, 0]
    h = _gelu(_ln(h, ln1g_ref[0, 0], ln1b_ref[0, 0]))
    r = jnp.dot(x, res_ref[0], preferred_element_type=jnp.float32) + resb_ref[0, 0]
    h = h + r
    h = jnp.dot(h, fc2_ref[0], preferred_element_type=jnp.float32) + fc2b_ref[0, 0]
    h = _gelu(_ln(h, ln2g_ref[0, 0], ln2b_ref[0, 0]))
    h = jnp.dot(h, fc3_ref[0], preferred_element_type=jnp.float32) + fc3b_ref[0, 0]
    h = _gelu(_ln(h, ln3g_ref[0, 0], ln3b_ref[0, 0]))
    o = jnp.dot(h, fc4_ref[0], preferred_element_type=jnp.float32) + fc4b_ref[0, 0]
    out_ref[...] = o * w_ref[:, 0:1]


def _grouped_mlp(gid, xs, ws, fc1_W, fc1_b, ln1_g, ln1_b, res_W, res_b,
                 fc2_W, fc2_b, ln2_g, ln2_b, fc3_W, fc3_b, ln3_g, ln3_b,
                 fc4_W, fc4_b):
    n_tiles = _P // _T

    def row_spec(cols):
        return pl.BlockSpec((_T, cols), lambda i, g: (i, 0))

    def w3_spec(r, c):
        return pl.BlockSpec((1, r, c), lambda i, g: (g[i], 0, 0))

    def w2_spec(c):
        return pl.BlockSpec((1, 1, c), lambda i, g: (g[i], 0, 0))

    grid_spec = pltpu.PrefetchScalarGridSpec(
        num_scalar_prefetch=1,
        grid=(n_tiles,),
        in_specs=[
            row_spec(_D),
            row_spec(128),
            w3_spec(_D, _H), w2_spec(_H), w2_spec(_H), w2_spec(_H),
            w3_spec(_D, _H), w2_spec(_H),
            w3_spec(_H, _H), w2_spec(_H), w2_spec(_H), w2_spec(_H),
            w3_spec(_H, _F), w2_spec(_F), w2_spec(_F), w2_spec(_F),
            w3_spec(_F, _O), w2_spec(_O),
        ],
        out_specs=row_spec(_O),
    )
    def v3(p):
        return p[:, None, :]

    return pl.pallas_call(
        _expert_body,
        grid_spec=grid_spec,
        out_shape=jax.ShapeDtypeStruct((_P, _O), jnp.float32),
    )(gid, xs, ws, fc1_W, v3(fc1_b), v3(ln1_g), v3(ln1_b), res_W, v3(res_b),
      fc2_W, v3(fc2_b), v3(ln2_g), v3(ln2_b), fc3_W, v3(fc3_b), v3(ln3_g),
      v3(ln3_b), fc4_W, v3(fc4_b))


def kernel(x, gate_W, gate_b, fc1_W, fc1_b, ln1_g, ln1_b, res_W, res_b,
           fc2_W, fc2_b, ln2_g, ln2_b, fc3_W, fc3_b, ln3_g, ln3_b,
           fc4_W, fc4_b):
    logits = x @ gate_W + gate_b
    probs = jax.nn.softmax(logits, axis=-1)
    topv, topi = jax.lax.top_k(probs, _K)
    wn = topv / (jnp.sum(topv, axis=-1, keepdims=True) + 1e-9)

    e_flat = topi.reshape(-1).astype(jnp.int32)
    order = jnp.argsort(e_flat, stable=True)
    se = e_flat[order]
    tok = (order // _K).astype(jnp.int32)
    sw = wn.reshape(-1)[order]

    counts = jnp.bincount(e_flat, length=_E)
    padded = ((counts + _T - 1) // _T) * _T
    pad_end = jnp.cumsum(padded)
    pad_off = pad_end - padded
    grp_start = jnp.cumsum(counts) - counts
    a = jnp.arange(_N * _K, dtype=jnp.int32)
    dest = (pad_off[se] + (a - grp_start[se])).astype(jnp.int32)

    gather_idx = jnp.zeros((_P,), jnp.int32).at[dest].set(tok)
    w_arr = jnp.zeros((_P,), jnp.float32).at[dest].set(sw)
    pos = jnp.zeros((_N * _K,), jnp.int32).at[order].set(dest).reshape(_N, _K)

    tile_start = jnp.arange(_P // _T, dtype=jnp.int32) * _T
    gid = jnp.searchsorted(pad_end, tile_start, side='right')
    gid = jnp.minimum(gid, _E - 1).astype(jnp.int32)

    xs = jnp.take(x, gather_idx, axis=0)
    ws = jnp.broadcast_to(w_arr[:, None], (_P, 128))

    out_sorted = _grouped_mlp(
        gid, xs, ws, fc1_W, fc1_b, ln1_g, ln1_b, res_W, res_b,
        fc2_W, fc2_b, ln2_g, ln2_b, fc3_W, fc3_b, ln3_g, ln3_b,
        fc4_W, fc4_b)

    y = jnp.take(out_sorted, pos[:, 0], axis=0) + jnp.take(out_sorted, pos[:, 1], axis=0)
    return y
